# baseline (speedup 1.0000x reference)
import functools

import jax
import jax.numpy as jnp
from jax import lax
from jax.experimental import pallas as pl
from jax.experimental.pallas import tpu as pltpu
from jax.experimental.pallas import tpu_sc as plsc

N = 10000
E = 320000
D = 128
NUM_LAYERS = 3

_SC_INFO = plsc.get_sparse_core_info()
NC = _SC_INFO.num_cores
NS = _SC_INFO.num_subcores
NW = NC * NS
EPW = E // NW
EB = 40
NBATCH = EPW // EB
CHUNK = 50
SUP = 6
NSUP = NBATCH // SUP
TILE_STRIDE = 624
TILE_ROWS = 640


def _edge_kernel_body(k_hbm, qv_hbm, eidx_hbm, out_hbm,
                      eidx, qvo, msg, aggr,
                      sem_gk0, sem_gk1, sem_gk2, sem_gq0, sem_gq1,
                      sem_s0, sem_s1, sem_s2):
    cid = lax.axis_index("c")
    sid = lax.axis_index("s")
    wid = cid * NS + sid
    sem_gk = (sem_gk0, sem_gk1, sem_gk2)
    sem_gq = (sem_gq0, sem_gq1)
    sem_s = (sem_s0, sem_s1, sem_s2)

    def _zrow(i, _):
        for j in range(D // 16):
            msg[0, i, pl.ds(j * 16, 16)] = jnp.zeros((16,), jnp.float32)
        return _
    lax.fori_loop(0, EB, _zrow, None)
    base_row = sid * TILE_STRIDE
    for z in range(TILE_ROWS // EB):
        pltpu.sync_copy(msg.at[0], aggr.at[pl.ds(base_row + z * EB, EB)])
    plsc.subcore_barrier()

    def _gissue(b, ms, qs):
        r = lax.rem(b, CHUNK)
        pltpu.async_copy(k_hbm.at[eidx.at[1, r]], msg.at[ms], sem_gk[ms])
        pltpu.async_copy(qv_hbm.at[eidx.at[0, r]], qvo.at[qs], sem_gq[qs])

    def _gwait(b, ms, qs):
        r = lax.rem(b, CHUNK)
        pltpu.make_async_copy(
            k_hbm.at[eidx.at[1, r]], msg.at[ms], sem_gk[ms]).wait()
        pltpu.make_async_copy(
            qv_hbm.at[eidx.at[0, r]], qvo.at[qs], sem_gq[qs]).wait()

    def _swait(ms):
        pltpu.make_async_copy(
            msg.at[ms], aggr.at[eidx.at[1, 0]], sem_s[ms]).wait()

    def _batch(b, u, i=None, unroll=4):
        ms, qs, sn, sp = u % 3, u % 2, (u + 1) % 3, (u + 2) % 3
        qs1 = (u + 1) % 2
        nxt = b + 1
        _gwait(b, ms, qs)

        if i is None:
            if b % CHUNK >= 2:
                _swait(sn)
        else:
            @pl.when(lax.rem(b, CHUNK) >= 2)
            def _():
                _swait(sn)

        if i is None:
            if nxt < NBATCH and nxt % CHUNK != 0:
                _gissue(nxt, sn, qs1)
        else:
            @pl.when(lax.rem(nxt, CHUNK) != 0)
            def _():
                _gissue(nxt, sn, qs1)

        ms_ref = msg.at[ms]
        qr = qvo.at[qs]

        @plsc.parallel_loop(0, EB, 1, unroll=unroll)
        def _edge(e):
            for j in range(D // 16):
                kc = ms_ref[e, pl.ds(j * 16, 16)]
                qc = qr[e, pl.ds(j * 16, 16)]
                vc = qr[e, pl.ds(D + j * 16, 16)]
                ms_ref[e, pl.ds(j * 16, 16)] = (
                    vc / (1.0 + jnp.exp(-(kc + qc))))

        r = lax.rem(b, CHUNK)
        pltpu.async_copy(ms_ref, aggr.at[eidx.at[1, r]], sem_s[ms], add=True)

        if i is None:
            if nxt < NBATCH and nxt % CHUNK == 0:
                _swait(sp)
                _swait(ms)
                pltpu.sync_copy(eidx_hbm.at[:, wid, nxt // CHUNK], eidx)
                _gissue(nxt, sn, qs1)
        else:
            @pl.when(lax.rem(nxt, CHUNK) == 0)
            def _():
                _swait(sp)
                _swait(ms)
                pltpu.sync_copy(
                    eidx_hbm.at[:, wid, lax.div(nxt, CHUNK)], eidx)
                _gissue(nxt, sn, qs1)

    pltpu.sync_copy(eidx_hbm.at[:, wid, 0], eidx)
    _gissue(0, 0, 0)

    def _sstep(i, _):
        b = i * SUP
        for u in range(SUP):
            _batch(b + u, u, i=i)
        return _

    lax.fori_loop(0, NSUP, _sstep, None)
    for t in range(NSUP * SUP, NBATCH):
        _batch(t, t % SUP, unroll=1)
    _swait((NBATCH - 2) % 3)
    _swait((NBATCH - 1) % 3)
    plsc.subcore_barrier()

    out_row = cid * N + sid * TILE_STRIDE
    pltpu.sync_copy(aggr.at[pl.ds(sid * TILE_STRIDE, TILE_ROWS)],
                    out_hbm.at[pl.ds(out_row, TILE_ROWS)])


@functools.partial(jax.jit, static_argnames=())
def _edge_stage(k, qv, eidx):
    mesh = plsc.VectorSubcoreMesh(core_axis_name="c", subcore_axis_name="s")
    f = pl.kernel(
        _edge_kernel_body,
        out_type=jax.ShapeDtypeStruct((NC * N, D), jnp.float32),
        mesh=mesh,
        scratch_types=[
            pltpu.VMEM((2, CHUNK, EB), jnp.int32),
            pltpu.VMEM((2, EB, 2 * D), jnp.float32),
            pltpu.VMEM((3, EB, D), jnp.float32),
            pltpu.VMEM_SHARED((N, D), jnp.float32),
            pltpu.SemaphoreType.DMA,
            pltpu.SemaphoreType.DMA,
            pltpu.SemaphoreType.DMA,
            pltpu.SemaphoreType.DMA,
            pltpu.SemaphoreType.DMA,
            pltpu.SemaphoreType.DMA,
            pltpu.SemaphoreType.DMA,
            pltpu.SemaphoreType.DMA,
        ],
    )
    return f(k, qv, eidx)


def _proj_body(x_ref, w_ref, b_ref, k_ref, qv_ref, s_ref):
    out = jnp.dot(x_ref[...], w_ref[...],
                  preferred_element_type=jnp.float32) + b_ref[...]
    k_ref[...] = out[:, :D]
    qv_ref[...] = out[:, D:3 * D]
    s_ref[...] = out[:, 3 * D:]


def _proj(x, w4, b4):
    blk = 1000
    grid = (N // blk,)
    return pl.pallas_call(
        _proj_body,
        grid=grid,
        in_specs=[
            pl.BlockSpec((blk, D), lambda i: (i, 0)),
            pl.BlockSpec((D, 4 * D), lambda i: (0, 0)),
            pl.BlockSpec((1, 4 * D), lambda i: (0, 0)),
        ],
        out_specs=[
            pl.BlockSpec((blk, D), lambda i: (i, 0)),
            pl.BlockSpec((blk, 2 * D), lambda i: (i, 0)),
            pl.BlockSpec((blk, D), lambda i: (i, 0)),
        ],
        out_shape=[
            jax.ShapeDtypeStruct((N, D), jnp.float32),
            jax.ShapeDtypeStruct((N, 2 * D), jnp.float32),
            jax.ShapeDtypeStruct((N, D), jnp.float32),
        ],
    )(x, w4, b4)


def _bn_proj_body(p_ref, s_ref, g_ref, bt_ref, w_ref, b_ref,
                  k_ref, qv_ref, sk_ref):
    h = p_ref[:N, :] + p_ref[N:, :] + s_ref[...]
    mu = jnp.mean(h, axis=0, keepdims=True)
    var = jnp.mean((h - mu) * (h - mu), axis=0, keepdims=True)
    hn = g_ref[...] * (h - mu) * lax.rsqrt(var + 1e-5) + bt_ref[...]
    h2 = jnp.maximum(hn, 0.0)
    out = jnp.dot(h2, w_ref[...],
                  preferred_element_type=jnp.float32) + b_ref[...]
    k_ref[...] = out[:, :D]
    qv_ref[...] = out[:, D:3 * D]
    sk_ref[...] = out[:, 3 * D:]


def _bn_proj(p, s, gamma, beta, w4, b4):
    return pl.pallas_call(
        _bn_proj_body,
        out_shape=[
            jax.ShapeDtypeStruct((N, D), jnp.float32),
            jax.ShapeDtypeStruct((N, 2 * D), jnp.float32),
            jax.ShapeDtypeStruct((N, D), jnp.float32),
        ],
    )(p, s, gamma, beta, w4, b4)


def _final_body(p_ref, s_ref, o_ref):
    o_ref[...] = p_ref[:N, :] + p_ref[N:, :] + s_ref[...]


def _final(p, s):
    return pl.pallas_call(
        _final_body,
        out_shape=jax.ShapeDtypeStruct((N, D), jnp.float32),
    )(p, s)


def kernel(x, edge_index, params):
    eidx = edge_index.reshape(2, NW, NBATCH // CHUNK, CHUNK, EB)

    w4s, b4s = [], []
    for p in params:
        w4s.append(jnp.concatenate(
            [p['Wk'], p['Wq'], p['Wv'], p['Ws']], axis=1))
        b4s.append(jnp.concatenate(
            [p['bk'], p['bq'], p['bv'], p['b']]).reshape(1, 4 * D))

    k, qv, s = _proj(x, w4s[0], b4s[0])
    for i in range(NUM_LAYERS):
        part = _edge_stage(k, qv, eidx)
        if i < NUM_LAYERS - 1:
            pr = params[i]
            k, qv, s = _bn_proj(part, s,
                                pr['gamma'].reshape(1, D),
                                pr['beta'].reshape(1, D),
                                w4s[i + 1], b4s[i + 1])
        else:
            return _final(part, s)


# speedup vs baseline: 1.0538x; 1.0538x over previous
import functools

import jax
import jax.numpy as jnp
from jax import lax
from jax.experimental import pallas as pl
from jax.experimental.pallas import tpu as pltpu
from jax.experimental.pallas import tpu_sc as plsc

N = 10000
E = 320000
D = 128
NUM_LAYERS = 3

_SC_INFO = plsc.get_sparse_core_info()
NC = _SC_INFO.num_cores
NS = _SC_INFO.num_subcores
NW = NC * NS
EPW = E // NW
EB = 40
NBATCH = EPW // EB
CHUNK = 50
NSTEP = NBATCH // 2
TILE_STRIDE = 624
TILE_ROWS = 640


def _edge_kernel_body(k_hbm, qv_hbm, eidx_hbm, out_hbm,
                      eidx, kro, qvo, msg, aggr,
                      sem_gk0, sem_gk1, sem_gq0, sem_gq1):
    cid = lax.axis_index("c")
    sid = lax.axis_index("s")
    wid = cid * NS + sid
    sem_gk = (sem_gk0, sem_gk1)
    sem_gq = (sem_gq0, sem_gq1)

    def _zrow(i, _):
        for j in range(D // 16):
            msg[i, pl.ds(j * 16, 16)] = jnp.zeros((16,), jnp.float32)
        return _
    lax.fori_loop(0, EB, _zrow, None)
    base_row = sid * TILE_STRIDE
    for z in range(TILE_ROWS // EB):
        pltpu.sync_copy(msg, aggr.at[pl.ds(base_row + z * EB, EB)])
    plsc.subcore_barrier()

    def _gissue(slot, b):
        r = lax.rem(b, CHUNK)
        pltpu.async_copy(k_hbm.at[eidx.at[1, r]], kro.at[slot], sem_gk[slot])
        pltpu.async_copy(qv_hbm.at[eidx.at[0, r]], qvo.at[slot], sem_gq[slot])

    def _gwait(slot, b):
        r = lax.rem(b, CHUNK)
        pltpu.make_async_copy(
            k_hbm.at[eidx.at[1, r]], kro.at[slot], sem_gk[slot]).wait()
        pltpu.make_async_copy(
            qv_hbm.at[eidx.at[0, r]], qvo.at[slot], sem_gq[slot]).wait()

    def _compute_scatter(slot, b):
        r = lax.rem(b, CHUNK)
        kr = kro.at[slot]
        qr = qvo.at[slot]

        @plsc.parallel_loop(0, EB, 1, unroll=4)
        def _edge(e):
            for j in range(D // 16):
                kc = kr[e, pl.ds(j * 16, 16)]
                qc = qr[e, pl.ds(j * 16, 16)]
                vc = qr[e, pl.ds(D + j * 16, 16)]
                msg[e, pl.ds(j * 16, 16)] = vc / (1.0 + jnp.exp(-(kc + qc)))

        pltpu.sync_copy(msg, aggr.at[eidx.at[1, r]], add=True)

    pltpu.sync_copy(eidx_hbm.at[:, wid, 0], eidx)
    _gissue(0, 0)

    def _dstep(i, _):
        b0 = 2 * i
        b1 = b0 + 1
        b2 = b0 + 2
        _gissue(1, b1)
        _gwait(0, b0)
        _compute_scatter(0, b0)

        crossing = lax.rem(i, CHUNK // 2) == CHUNK // 2 - 1

        @pl.when(jnp.logical_and(jnp.logical_not(crossing), b2 < NBATCH))
        def _():
            _gissue(0, b2)

        _gwait(1, b1)
        _compute_scatter(1, b1)

        @pl.when(jnp.logical_and(crossing, b2 < NBATCH))
        def _():
            pltpu.sync_copy(eidx_hbm.at[:, wid, lax.div(b2, CHUNK)], eidx)
            _gissue(0, b2)
        return _

    lax.fori_loop(0, NSTEP, _dstep, None)
    plsc.subcore_barrier()

    out_row = cid * N + sid * TILE_STRIDE
    pltpu.sync_copy(aggr.at[pl.ds(sid * TILE_STRIDE, TILE_ROWS)],
                    out_hbm.at[pl.ds(out_row, TILE_ROWS)])


@functools.partial(jax.jit, static_argnames=())
def _edge_stage(k, qv, eidx):
    mesh = plsc.VectorSubcoreMesh(core_axis_name="c", subcore_axis_name="s")
    f = pl.kernel(
        _edge_kernel_body,
        out_type=jax.ShapeDtypeStruct((NC * N, D), jnp.float32),
        mesh=mesh,
        scratch_types=[
            pltpu.VMEM((2, CHUNK, EB), jnp.int32),
            pltpu.VMEM((2, EB, D), jnp.float32),
            pltpu.VMEM((2, EB, 2 * D), jnp.float32),
            pltpu.VMEM((EB, D), jnp.float32),
            pltpu.VMEM_SHARED((N, D), jnp.float32),
            pltpu.SemaphoreType.DMA,
            pltpu.SemaphoreType.DMA,
            pltpu.SemaphoreType.DMA,
            pltpu.SemaphoreType.DMA,
        ],
    )
    return f(k, qv, eidx)


def _proj_body(x_ref, w_ref, b_ref, k_ref, qv_ref, s_ref):
    out = jnp.dot(x_ref[...], w_ref[...],
                  preferred_element_type=jnp.float32) + b_ref[...]
    k_ref[...] = out[:, :D]
    qv_ref[...] = out[:, D:3 * D]
    s_ref[...] = out[:, 3 * D:]


def _proj(x, w4, b4):
    blk = 1000
    grid = (N // blk,)
    return pl.pallas_call(
        _proj_body,
        grid=grid,
        in_specs=[
            pl.BlockSpec((blk, D), lambda i: (i, 0)),
            pl.BlockSpec((D, 4 * D), lambda i: (0, 0)),
            pl.BlockSpec((1, 4 * D), lambda i: (0, 0)),
        ],
        out_specs=[
            pl.BlockSpec((blk, D), lambda i: (i, 0)),
            pl.BlockSpec((blk, 2 * D), lambda i: (i, 0)),
            pl.BlockSpec((blk, D), lambda i: (i, 0)),
        ],
        out_shape=[
            jax.ShapeDtypeStruct((N, D), jnp.float32),
            jax.ShapeDtypeStruct((N, 2 * D), jnp.float32),
            jax.ShapeDtypeStruct((N, D), jnp.float32),
        ],
    )(x, w4, b4)


def _bn_proj_body(p_ref, s_ref, g_ref, bt_ref, w_ref, b_ref,
                  k_ref, qv_ref, sk_ref):
    h = p_ref[:N, :] + p_ref[N:, :] + s_ref[...]
    mu = jnp.mean(h, axis=0, keepdims=True)
    var = jnp.mean((h - mu) * (h - mu), axis=0, keepdims=True)
    hn = g_ref[...] * (h - mu) * lax.rsqrt(var + 1e-5) + bt_ref[...]
    h2 = jnp.maximum(hn, 0.0)
    out = jnp.dot(h2, w_ref[...],
                  preferred_element_type=jnp.float32) + b_ref[...]
    k_ref[...] = out[:, :D]
    qv_ref[...] = out[:, D:3 * D]
    sk_ref[...] = out[:, 3 * D:]


def _bn_proj(p, s, gamma, beta, w4, b4):
    return pl.pallas_call(
        _bn_proj_body,
        out_shape=[
            jax.ShapeDtypeStruct((N, D), jnp.float32),
            jax.ShapeDtypeStruct((N, 2 * D), jnp.float32),
            jax.ShapeDtypeStruct((N, D), jnp.float32),
        ],
    )(p, s, gamma, beta, w4, b4)


def _final_body(p_ref, s_ref, o_ref):
    o_ref[...] = p_ref[:N, :] + p_ref[N:, :] + s_ref[...]


def _final(p, s):
    return pl.pallas_call(
        _final_body,
        out_shape=jax.ShapeDtypeStruct((N, D), jnp.float32),
    )(p, s)


def kernel(x, edge_index, params):
    eidx = edge_index.reshape(2, NW, NBATCH // CHUNK, CHUNK, EB)

    w4s, b4s = [], []
    for p in params:
        w4s.append(jnp.concatenate(
            [p['Wk'], p['Wq'], p['Wv'], p['Ws']], axis=1))
        b4s.append(jnp.concatenate(
            [p['bk'], p['bq'], p['bv'], p['b']]).reshape(1, 4 * D))

    k, qv, s = _proj(x, w4s[0], b4s[0])
    for i in range(NUM_LAYERS):
        part = _edge_stage(k, qv, eidx)
        if i < NUM_LAYERS - 1:
            pr = params[i]
            k, qv, s = _bn_proj(part, s,
                                pr['gamma'].reshape(1, D),
                                pr['beta'].reshape(1, D),
                                w4s[i + 1], b4s[i + 1])
        else:
            return _final(part, s)
